# Initial kernel scaffold; baseline (speedup 1.0000x reference)
import jax
import jax.numpy as jnp
from jax.experimental import pallas as pl


def kernel(x, edge_index, W1, b1, W2, b2, W3, b3):
    raise NotImplementedError("write your pallas kernel here")


# speedup vs baseline: 17.5390x; 17.5390x over previous
import functools

import jax
import jax.numpy as jnp
from jax import lax
from jax.experimental import pallas as pl
from jax.experimental.pallas import tpu as pltpu
from jax.experimental.pallas import tpu_sc as plsc

_N = 10000
_NF = 128
_H1, _H2, _H3 = 64, 32, 40
_E = 320000

_NSUB = 16
_NTILE = 32
_CH = 128
_NCH = 79
_EP = _CH * _NCH
_EPAD = _NTILE * _EP
_NPAD = 10240
_RPS = _NPAD // _NSUB


def _sc_mesh():
    return plsc.VectorSubcoreMesh(
        core_axis_name="c", subcore_axis_name="s", num_cores=2, num_subcores=_NSUB
    )


def _deg_body(dst_hbm, zeros_hbm, ones_hbm, out_hbm, dst_v, buf, acc):
    c = lax.axis_index("c")
    s = lax.axis_index("s")
    pltpu.sync_copy(dst_hbm.at[c * _NSUB + s], dst_v)
    pltpu.sync_copy(ones_hbm, buf)
    pltpu.sync_copy(zeros_hbm, acc.at[pl.ds(s * _RPS, _RPS)])
    plsc.subcore_barrier()

    def body(j, carry):
        pltpu.sync_copy(buf, acc.at[dst_v.at[j]], add=True)
        return carry

    lax.fori_loop(0, _NCH, body, 0)
    plsc.subcore_barrier()
    pltpu.sync_copy(acc.at[pl.ds(s * _RPS, _RPS)], out_hbm.at[c].at[pl.ds(s * _RPS, _RPS)])


def _deg_call(dst3, zeros_2d, ones_2d):
    kern = pl.kernel(
        _deg_body,
        out_type=jax.ShapeDtypeStruct((2, _NPAD, 1), jnp.float32),
        mesh=_sc_mesh(),
        compiler_params=pltpu.CompilerParams(use_tc_tiling_on_sc=False),
        scratch_types=[
            pltpu.VMEM((_NCH, _CH), jnp.int32),
            pltpu.VMEM((_CH, 1), jnp.float32),
            pltpu.VMEM_SHARED((_NPAD, 1), jnp.float32),
        ],
    )
    return kern(dst3, zeros_2d, ones_2d)


def _agg_body(g_hbm, src_hbm, dst_hbm, zeros_hbm, out_hbm, src_v, dst_v, buf, acc, gsem):
    c = lax.axis_index("c")
    s = lax.axis_index("s")
    tid = c * _NSUB + s
    pltpu.sync_copy(src_hbm.at[tid], src_v)
    pltpu.sync_copy(dst_hbm.at[tid], dst_v)
    pltpu.sync_copy(zeros_hbm, acc.at[pl.ds(s * _RPS, _RPS)])
    plsc.subcore_barrier()

    def body(j, carry):
        pltpu.async_copy(g_hbm.at[src_v.at[j]], buf, gsem).wait()
        pltpu.sync_copy(buf, acc.at[dst_v.at[j]], add=True)
        return carry

    lax.fori_loop(0, _NCH, body, 0)
    plsc.subcore_barrier()
    pltpu.sync_copy(acc.at[pl.ds(s * _RPS, _RPS)], out_hbm.at[c].at[pl.ds(s * _RPS, _RPS)])


def _agg_call(f, g, src3, dst3, zeros_2d):
    kern = pl.kernel(
        _agg_body,
        out_type=jax.ShapeDtypeStruct((2, _NPAD, f), jnp.float32),
        mesh=_sc_mesh(),
        compiler_params=pltpu.CompilerParams(use_tc_tiling_on_sc=False),
        scratch_types=[
            pltpu.VMEM((_NCH, _CH), jnp.int32),
            pltpu.VMEM((_NCH, _CH), jnp.int32),
            pltpu.VMEM((_CH, f), jnp.float32),
            pltpu.VMEM_SHARED((_NPAD, f), jnp.float32),
            pltpu.SemaphoreType.DMA,
        ],
    )
    return kern(g, src3, dst3, zeros_2d)


def _dinv_from(deg_ref):
    deg = jnp.sum(deg_ref[...], axis=1, keepdims=True)[:_N] + 1.0
    return lax.rsqrt(deg)


def _tc1_body(deg_ref, x_ref, w_ref, g_ref):
    dinv = _dinv_from(deg_ref)
    h = jnp.dot(x_ref[...], w_ref[...], preferred_element_type=jnp.float32)
    g_ref[...] = h * dinv


def _tc2_body(deg_ref, s_ref, gin_ref, b_ref, w_ref, g_ref):
    dinv = _dinv_from(deg_ref)
    sagg = s_ref[0, : _N, :] + s_ref[1, : _N, :]
    a = jnp.tanh(dinv * (sagg + gin_ref[...]) + b_ref[...])
    g_ref[...] = jnp.dot(a, w_ref[...], preferred_element_type=jnp.float32) * dinv


def _tc3_body(deg_ref, s_ref, gin_ref, b_ref, w_ref, emb_ref, g_ref):
    dinv = _dinv_from(deg_ref)
    sagg = s_ref[0, : _N, :] + s_ref[1, : _N, :]
    emb = dinv * (sagg + gin_ref[...]) + b_ref[...]
    emb_ref[...] = emb
    a = jnp.tanh(emb)
    g_ref[...] = jnp.dot(a, w_ref[...], preferred_element_type=jnp.float32) * dinv


def _tc4_body(deg_ref, s_ref, gin_ref, b_ref, out_ref):
    dinv = _dinv_from(deg_ref)
    sagg = s_ref[0, : _N, :] + s_ref[1, : _N, :]
    out_ref[...] = dinv * (sagg + gin_ref[...]) + b_ref[...]


def _tc1(degT, x, w):
    return pl.pallas_call(
        _tc1_body, out_shape=jax.ShapeDtypeStruct((_N, _H1), jnp.float32)
    )(degT, x, w)


def _tc2(degT, s1, g1, b1, w2):
    return pl.pallas_call(
        _tc2_body, out_shape=jax.ShapeDtypeStruct((_N, _H2), jnp.float32)
    )(degT, s1, g1, b1, w2)


def _tc3(degT, s2, g2, b2, w3):
    return pl.pallas_call(
        _tc3_body,
        out_shape=[
            jax.ShapeDtypeStruct((_N, _H2), jnp.float32),
            jax.ShapeDtypeStruct((_N, _H3), jnp.float32),
        ],
    )(degT, s2, g2, b2, w3)


def _tc4(degT, s3, g3, b3):
    return pl.pallas_call(
        _tc4_body, out_shape=jax.ShapeDtypeStruct((_N, _H3), jnp.float32)
    )(degT, s3, g3, b3)


def kernel(x, edge_index, W1, b1, W2, b2, W3, b3):
    src = edge_index[0]
    dst = edge_index[1]
    padlen = _EPAD - _E
    srcp = jnp.concatenate([src, jnp.zeros((padlen,), src.dtype)])
    dstp = jnp.concatenate([dst, jnp.full((padlen,), _N, dst.dtype)])
    src3 = srcp.reshape(_NTILE, _NCH, _CH)
    dst3 = dstp.reshape(_NTILE, _NCH, _CH)

    deg_parts = _deg_call(
        dst3, jnp.zeros((_RPS, 1), jnp.float32), jnp.ones((_CH, 1), jnp.float32)
    )
    degT = deg_parts.reshape(2, _NPAD).T

    g1 = _tc1(degT, x, W1)
    s1 = _agg_call(_H1, g1, src3, dst3, jnp.zeros((_RPS, _H1), jnp.float32))
    g2 = _tc2(degT, s1, g1, b1.reshape(1, -1), W2)
    s2 = _agg_call(_H2, g2, src3, dst3, jnp.zeros((_RPS, _H2), jnp.float32))
    emb, g3 = _tc3(degT, s2, g2, b2.reshape(1, -1), W3)
    s3 = _agg_call(_H3, g3, src3, dst3, jnp.zeros((_RPS, _H3), jnp.float32))
    logits = _tc4(degT, s3, g3, b3.reshape(1, -1))
    return (logits, emb)


# speedup vs baseline: 21.5777x; 1.2303x over previous
import functools

import jax
import jax.numpy as jnp
from jax import lax
from jax.experimental import pallas as pl
from jax.experimental.pallas import tpu as pltpu
from jax.experimental.pallas import tpu_sc as plsc

_N = 10000
_NF = 128
_H1, _H2, _H3 = 64, 32, 40
_E = 320000

_NSUB = 16
_NTILE = 32
_CH = 128
_NCH = 79
_EP = _CH * _NCH
_EPAD = _NTILE * _EP
_NPAD = 10240
_RPS = _NPAD // _NSUB


def _sc_mesh():
    return plsc.VectorSubcoreMesh(
        core_axis_name="c", subcore_axis_name="s", num_cores=2, num_subcores=_NSUB
    )


def _deg_body(dst_hbm, zeros_hbm, ones_hbm, out_hbm, dst_v, buf, acc):
    c = lax.axis_index("c")
    s = lax.axis_index("s")
    pltpu.sync_copy(dst_hbm.at[c * _NSUB + s], dst_v)
    pltpu.sync_copy(ones_hbm, buf)
    pltpu.sync_copy(zeros_hbm, acc.at[pl.ds(s * _RPS, _RPS)])
    plsc.subcore_barrier()

    def body(j, carry):
        pltpu.sync_copy(buf, acc.at[dst_v.at[j]], add=True)
        return carry

    lax.fori_loop(0, _NCH, body, 0)
    plsc.subcore_barrier()
    pltpu.sync_copy(acc.at[pl.ds(s * _RPS, _RPS)], out_hbm.at[c].at[pl.ds(s * _RPS, _RPS)])


def _deg_call(dst3, zeros_2d, ones_2d):
    kern = pl.kernel(
        _deg_body,
        out_type=jax.ShapeDtypeStruct((2, _NPAD, 1), jnp.float32),
        mesh=_sc_mesh(),
        compiler_params=pltpu.CompilerParams(use_tc_tiling_on_sc=False),
        scratch_types=[
            pltpu.VMEM((_NCH, _CH), jnp.int32),
            pltpu.VMEM((_CH, 1), jnp.float32),
            pltpu.VMEM_SHARED((_NPAD, 1), jnp.float32),
        ],
    )
    return kern(dst3, zeros_2d, ones_2d)


def _agg_body(g_hbm, src_hbm, dst_hbm, zeros_hbm, out_hbm, src_v, dst_v, buf, acc, gsem):
    c = lax.axis_index("c")
    s = lax.axis_index("s")
    tid = c * _NSUB + s
    pltpu.sync_copy(src_hbm.at[tid], src_v)
    pltpu.sync_copy(dst_hbm.at[tid], dst_v)
    pltpu.sync_copy(zeros_hbm, acc.at[pl.ds(s * _RPS, _RPS)])
    plsc.subcore_barrier()

    pltpu.async_copy(g_hbm.at[src_v.at[0]], buf.at[0], gsem.at[0])

    def body(j, carry):
        slot = lax.rem(j, 2)
        nslot = lax.rem(j + 1, 2)

        @pl.when(j + 1 < _NCH)
        def _():
            pltpu.async_copy(g_hbm.at[src_v.at[j + 1]], buf.at[nslot], gsem.at[nslot])

        pltpu.make_async_copy(g_hbm.at[src_v.at[j]], buf.at[slot], gsem.at[slot]).wait()
        pltpu.sync_copy(buf.at[slot], acc.at[dst_v.at[j]], add=True)
        return carry

    lax.fori_loop(0, _NCH, body, 0)
    plsc.subcore_barrier()
    pltpu.sync_copy(acc.at[pl.ds(s * _RPS, _RPS)], out_hbm.at[c].at[pl.ds(s * _RPS, _RPS)])


def _agg_call(f, g, src3, dst3, zeros_2d):
    kern = pl.kernel(
        _agg_body,
        out_type=jax.ShapeDtypeStruct((2, _NPAD, f), jnp.float32),
        mesh=_sc_mesh(),
        compiler_params=pltpu.CompilerParams(use_tc_tiling_on_sc=False),
        scratch_types=[
            pltpu.VMEM((_NCH, _CH), jnp.int32),
            pltpu.VMEM((_NCH, _CH), jnp.int32),
            pltpu.VMEM((2, _CH, f), jnp.float32),
            pltpu.VMEM_SHARED((_NPAD, f), jnp.float32),
            pltpu.SemaphoreType.DMA((2,)),
        ],
    )
    return kern(g, src3, dst3, zeros_2d)


def _dinv_from(deg_ref):
    deg = jnp.sum(deg_ref[...], axis=1, keepdims=True)[:_N] + 1.0
    return lax.rsqrt(deg)


def _tc1_body(deg_ref, x_ref, w_ref, g_ref):
    dinv = _dinv_from(deg_ref)
    h = jnp.dot(x_ref[...], w_ref[...], preferred_element_type=jnp.float32)
    g_ref[...] = h * dinv


def _tc2_body(deg_ref, s_ref, gin_ref, b_ref, w_ref, g_ref):
    dinv = _dinv_from(deg_ref)
    sagg = s_ref[0, : _N, :] + s_ref[1, : _N, :]
    a = jnp.tanh(dinv * (sagg + gin_ref[...]) + b_ref[...])
    g_ref[...] = jnp.dot(a, w_ref[...], preferred_element_type=jnp.float32) * dinv


def _tc3_body(deg_ref, s_ref, gin_ref, b_ref, w_ref, emb_ref, g_ref):
    dinv = _dinv_from(deg_ref)
    sagg = s_ref[0, : _N, :] + s_ref[1, : _N, :]
    emb = dinv * (sagg + gin_ref[...]) + b_ref[...]
    emb_ref[...] = emb
    a = jnp.tanh(emb)
    g_ref[...] = jnp.dot(a, w_ref[...], preferred_element_type=jnp.float32) * dinv


def _tc4_body(deg_ref, s_ref, gin_ref, b_ref, out_ref):
    dinv = _dinv_from(deg_ref)
    sagg = s_ref[0, : _N, :] + s_ref[1, : _N, :]
    out_ref[...] = dinv * (sagg + gin_ref[...]) + b_ref[...]


def _tc1(degT, x, w):
    return pl.pallas_call(
        _tc1_body, out_shape=jax.ShapeDtypeStruct((_N, _H1), jnp.float32)
    )(degT, x, w)


def _tc2(degT, s1, g1, b1, w2):
    return pl.pallas_call(
        _tc2_body, out_shape=jax.ShapeDtypeStruct((_N, _H2), jnp.float32)
    )(degT, s1, g1, b1, w2)


def _tc3(degT, s2, g2, b2, w3):
    return pl.pallas_call(
        _tc3_body,
        out_shape=[
            jax.ShapeDtypeStruct((_N, _H2), jnp.float32),
            jax.ShapeDtypeStruct((_N, _H3), jnp.float32),
        ],
    )(degT, s2, g2, b2, w3)


def _tc4(degT, s3, g3, b3):
    return pl.pallas_call(
        _tc4_body, out_shape=jax.ShapeDtypeStruct((_N, _H3), jnp.float32)
    )(degT, s3, g3, b3)


def kernel(x, edge_index, W1, b1, W2, b2, W3, b3):
    src = edge_index[0]
    dst = edge_index[1]
    padlen = _EPAD - _E
    srcp = jnp.concatenate([src, jnp.zeros((padlen,), src.dtype)])
    dstp = jnp.concatenate([dst, jnp.full((padlen,), _N, dst.dtype)])
    src3 = srcp.reshape(_NTILE, _NCH, _CH)
    dst3 = dstp.reshape(_NTILE, _NCH, _CH)

    deg_parts = _deg_call(
        dst3, jnp.zeros((_RPS, 1), jnp.float32), jnp.ones((_CH, 1), jnp.float32)
    )
    degT = deg_parts.reshape(2, _NPAD).T

    g1 = _tc1(degT, x, W1)
    s1 = _agg_call(_H1, g1, src3, dst3, jnp.zeros((_RPS, _H1), jnp.float32))
    g2 = _tc2(degT, s1, g1, b1.reshape(1, -1), W2)
    s2 = _agg_call(_H2, g2, src3, dst3, jnp.zeros((_RPS, _H2), jnp.float32))
    emb, g3 = _tc3(degT, s2, g2, b2.reshape(1, -1), W3)
    s3 = _agg_call(_H3, g3, src3, dst3, jnp.zeros((_RPS, _H3), jnp.float32))
    logits = _tc4(degT, s3, g3, b3.reshape(1, -1))
    return (logits, emb)
